# Initial kernel scaffold; baseline (speedup 1.0000x reference)
import jax
import jax.numpy as jnp
from jax.experimental import pallas as pl


def kernel(concepts_pred_logits, concepts_true, min_distances, proto_presence, target_pred_logits, target_true, encoder_proto_presence, last_layer_weight, l1_mask):
    raise NotImplementedError("write your pallas kernel here")


# speedup vs baseline: 72.0780x; 72.0780x over previous
import functools

import jax
import jax.numpy as jnp
from jax import lax
from jax.experimental import pallas as pl
from jax.experimental.pallas import tpu as pltpu

B = 128
C = 112
P = 256
D = 10
NC = 200
MAX_DIST = 128.0
ALPHA = 1.0
COEFS = {"crs_ent": 1.0, "clst": 0.8, "sep": -0.08, "l1": 1e-4,
         "ortho_p": 1.0, "ortho_c": 1.0}
EPS = 1e-8
R2 = 2 * C
NEG = -1e9


def _s_body(ppt_ref, s_ref):
    acc = ppt_ref[0]
    for d in range(1, D):
        acc = acc + ppt_ref[d]
    s_ref[...] = acc


def _dist_body(srow_ref, scol_ref, w1_ref, md_ref, acc1_ref, acc2_ref):
    r = pl.program_id(0)
    srow = srow_ref[0]
    scol = scol_ref[0]
    qi = lax.broadcasted_iota(jnp.int32, (P, P), 0)
    pi = lax.broadcasted_iota(jnp.int32, (P, P), 1)
    gt = (scol > srow).astype(jnp.float32)
    eqlt = jnp.logical_and(scol == srow, qi < pi).astype(jnp.float32)
    cnt = jnp.sum(gt + eqlt, axis=0, keepdims=True)
    in_top = cnt < float(D)

    v = MAX_DIST - md_ref[...]
    mx1 = jnp.max(jnp.where(in_top, v, NEG), axis=1, keepdims=True)
    mx2 = jnp.max(jnp.where(in_top, NEG, v), axis=1, keepdims=True)
    m1 = jnp.maximum(mx1, 0.0)
    m2 = jnp.maximum(mx2, 0.0)

    w1 = w1_ref[0]
    c1 = jnp.sum(w1 * m1)
    c2 = jnp.sum((1.0 - w1) * m2)

    @pl.when(r == 0)
    def _():
        acc1_ref[...] = jnp.zeros_like(acc1_ref)
        acc2_ref[...] = jnp.zeros_like(acc2_ref)

    acc1_ref[...] += c1
    acc2_ref[...] += c2


def _dense_body(cpl_ref, ctf_ref, eppt_ref, tpl_ref, tt_ref, w_ref, l1m_ref,
                acc1_ref, acc2_ref, scal_ref, closs_ref, rn_ref):
    x = cpl_ref[...]
    ct = ctf_ref[...]
    bce = (jnp.maximum(x, 0.0) - x * ct
           + jnp.log(1.0 + jnp.exp(-jnp.abs(x))))
    closs_ref[...] = jnp.mean(bce, axis=0, keepdims=True)
    entropy = jnp.mean(bce)

    def rn_body(d, _):
        xd = eppt_ref[d]
        na = jnp.sqrt(jnp.sum(xd * xd, axis=1, keepdims=True))
        rn_ref[d] = 1.0 / jnp.maximum(na, EPS)
        return 0
    lax.fori_loop(0, D, rn_body, 0)

    def cos_body(k, carry):
        ap, ac = carry
        d = k // D
        e = k - d * D
        xd = eppt_ref[d]
        xe = eppt_ref[e]
        rnd = rn_ref[d]
        rne = rn_ref[e]
        s_de = jnp.sum(xd * xe, axis=1, keepdims=True)
        ap = ap + jnp.sum(s_de * rnd * rne)
        s_c = jnp.sum(xd[C:, :] * xe[:C, :], axis=1, keepdims=True)
        ac = ac + jnp.sum(s_c * rnd[C:] * rne[:C])
        return ap, ac
    accp, accc = lax.fori_loop(0, D * D, cos_body, (0.0, 0.0))
    ortho_p = accp / (D * C * 2) - 1.0
    ortho_c = accc / (D * C)

    tl = tpl_ref[...]
    m = jnp.max(tl, axis=1, keepdims=True)
    lse = m + jnp.log(jnp.sum(jnp.exp(tl - m), axis=1, keepdims=True))
    lbl = tt_ref[...]
    onehot = lax.broadcasted_iota(jnp.int32, (B, NC), 1) == lbl
    picked = jnp.sum(jnp.where(onehot, tl, 0.0), axis=1, keepdims=True)
    target_loss = jnp.mean(lse - picked)

    l1 = jnp.sum(jnp.abs(w_ref[...] * l1m_ref[...]))

    clst = MAX_DIST - acc1_ref[0, 0] / (B * C)
    sep = MAX_DIST - acc2_ref[0, 0] / (B * C)

    summed = (COEFS["crs_ent"] * entropy + COEFS["clst"] * clst
              + COEFS["sep"] * sep + COEFS["l1"] * l1
              + COEFS["ortho_p"] * ortho_p + COEFS["ortho_c"] * ortho_c)
    total = target_loss + ALPHA * summed

    li = lax.broadcasted_iota(jnp.int32, (1, 16), 1)
    vals = [target_loss, summed, total, entropy, clst, sep, l1,
            ortho_p, ortho_c]
    out = jnp.zeros((1, 16), jnp.float32)
    for i, val in enumerate(vals):
        out = jnp.where(li == i, val, out)
    scal_ref[...] = out


@jax.jit
def kernel(concepts_pred_logits, concepts_true, min_distances, proto_presence,
           target_pred_logits, target_true, encoder_proto_presence,
           last_layer_weight, l1_mask):
    ctf = concepts_true.astype(jnp.float32)
    ppt = jnp.transpose(proto_presence, (2, 0, 1))
    eppt = jnp.transpose(encoder_proto_presence, (2, 0, 1))
    w1t = jnp.concatenate([ctf, 1.0 - ctf], axis=1).T
    tt2 = target_true.astype(jnp.int32).reshape(B, 1)

    s = pl.pallas_call(
        _s_body,
        out_shape=jax.ShapeDtypeStruct((R2, P), jnp.float32),
    )(ppt)

    srow = s.reshape(R2, 1, P)
    scol = s.reshape(R2, P, 1)
    w1t3 = w1t.reshape(R2, B, 1)

    acc1, acc2 = pl.pallas_call(
        _dist_body,
        grid=(R2,),
        in_specs=[
            pl.BlockSpec((1, 1, P), lambda r: (r, 0, 0)),
            pl.BlockSpec((1, P, 1), lambda r: (r, 0, 0)),
            pl.BlockSpec((1, B, 1), lambda r: (r, 0, 0)),
            pl.BlockSpec((B, P), lambda r: (0, 0)),
        ],
        out_specs=[
            pl.BlockSpec((1, 1), lambda r: (0, 0)),
            pl.BlockSpec((1, 1), lambda r: (0, 0)),
        ],
        out_shape=[
            jax.ShapeDtypeStruct((1, 1), jnp.float32),
            jax.ShapeDtypeStruct((1, 1), jnp.float32),
        ],
    )(srow, scol, w1t3, min_distances)

    scal, closs = pl.pallas_call(
        _dense_body,
        out_shape=[
            jax.ShapeDtypeStruct((1, 16), jnp.float32),
            jax.ShapeDtypeStruct((1, C), jnp.float32),
        ],
        scratch_shapes=[pltpu.VMEM((D, R2, 1), jnp.float32)],
    )(concepts_pred_logits, ctf, eppt, target_pred_logits, tt2,
      last_layer_weight, l1_mask, acc1, acc2)

    return (scal[0, 0], closs[0], scal[0, 1], scal[0, 2], scal[0, 3],
            scal[0, 4], scal[0, 5], scal[0, 6], scal[0, 7], scal[0, 8])


# speedup vs baseline: 127.7085x; 1.7718x over previous
import functools

import jax
import jax.numpy as jnp
from jax import lax
from jax.experimental import pallas as pl
from jax.experimental.pallas import tpu as pltpu
from jax.experimental.pallas import tpu_sc as plsc

B = 128
C = 112
P = 256
D = 10
NC = 200
MAX_DIST = 128.0
ALPHA = 1.0
COEFS = {"crs_ent": 1.0, "clst": 0.8, "sep": -0.08, "l1": 1e-4,
         "ortho_p": 1.0, "ortho_c": 1.0}
EPS = 1e-8
R2 = 2 * C
L = 16
NW = 32
RPW = R2 // NW
NBLK = P // L
NBB = B // L
BIGF = 3.0e38
NEGF = -3.0e38


def _sc_dist_body(mdt_hbm, ppf_hbm, wt_hbm, out1_hbm, out2_hbm,
                  mdv, ppv, wv, sv, a1v, a2v):
    cid = lax.axis_index("c")
    sid = lax.axis_index("s")
    wid = sid * 2 + cid
    pltpu.sync_copy(mdt_hbm, mdv)
    pltpu.sync_copy(ppf_hbm.at[pl.ds(wid * RPW * P * D, RPW * P * D)], ppv)
    pltpu.sync_copy(wt_hbm.at[pl.ds(wid * RPW * B, RPW * B)], wv)

    iota = lax.iota(jnp.int32, L)
    zi = jnp.zeros((L,), jnp.int32)

    def row_body(i, accs):
        acc1, acc2 = accs
        pbase = i * (P * D)
        s_chunks = []
        for blk in range(NBLK):
            gbase = pbase + blk * (L * D)
            idx0 = iota * D + gbase
            acc = plsc.load_gather(ppv, [idx0])
            for d in range(1, D):
                acc = acc + plsc.load_gather(ppv, [idx0 + d])
            s_chunks.append(acc)

        top_idx = []
        for _ in range(D):
            m = s_chunks[0]
            for blk in range(1, NBLK):
                m = jnp.maximum(m, s_chunks[blk])
            gm = jnp.max(m)
            cand = jnp.where(s_chunks[0] == gm, iota, jnp.int32(P))
            for blk in range(1, NBLK):
                cblk = jnp.where(s_chunks[blk] == gm, iota + blk * L,
                                 jnp.int32(P))
                cand = jnp.minimum(cand, cblk)
            idx = jnp.min(cand)
            top_idx.append(idx)
            for blk in range(NBLK):
                hit = (iota + blk * L) == idx
                s_chunks[blk] = jnp.where(hit, jnp.float32(NEGF),
                                          s_chunks[blk])

        def patch_body(k, _):
            b0 = k * L
            for j in range(D):
                sv[j, pl.ds(b0, L)] = mdv[top_idx[j], pl.ds(b0, L)]
                mdv[top_idx[j], pl.ds(b0, L)] = jnp.full((L,), BIGF,
                                                         jnp.float32)
            return 0
        lax.fori_loop(0, NBB, patch_body, 0)

        def bb_body(k, accs2):
            a1, a2 = accs2
            b0 = k * L

            def p_body(q, mn_c):
                p0 = q * 8
                for j in range(8):
                    mn_c = jnp.minimum(mn_c, mdv[p0 + j, pl.ds(b0, L)])
                return mn_c
            mn = lax.fori_loop(0, P // 8, p_body,
                               jnp.full((L,), BIGF, jnp.float32))
            m2 = jnp.maximum(MAX_DIST - mn, 0.0)

            mt = sv[0, pl.ds(b0, L)]
            for j in range(1, D):
                mt = jnp.minimum(mt, sv[j, pl.ds(b0, L)])
            m1 = jnp.maximum(MAX_DIST - mt, 0.0)

            w = wv[pl.ds(i * B + b0, L)]
            return (a1 + w * m1, a2 + (1.0 - w) * m2)
        acc1, acc2 = lax.fori_loop(0, NBB, bb_body, (acc1, acc2))

        def rest_body(k, _):
            b0 = k * L
            for j in range(D):
                mdv[top_idx[j], pl.ds(b0, L)] = sv[j, pl.ds(b0, L)]
            return 0
        lax.fori_loop(0, NBB, rest_body, 0)
        return (acc1, acc2)

    zero = jnp.zeros((L,), jnp.float32)
    acc1, acc2 = lax.fori_loop(0, RPW, row_body, (zero, zero))
    a1v[...] = acc1
    a2v[...] = acc2
    pltpu.sync_copy(a1v, out1_hbm.at[wid])
    pltpu.sync_copy(a2v, out2_hbm.at[wid])


_sc_dist = functools.partial(
    pl.kernel,
    mesh=plsc.VectorSubcoreMesh(core_axis_name="c", subcore_axis_name="s"),
    compiler_params=pltpu.CompilerParams(needs_layout_passes=False),
    out_type=[jax.ShapeDtypeStruct((NW, L), jnp.float32),
              jax.ShapeDtypeStruct((NW, L), jnp.float32)],
    scratch_types=[
        pltpu.VMEM((P, B), jnp.float32),
        pltpu.VMEM((RPW * P * D,), jnp.float32),
        pltpu.VMEM((RPW * B,), jnp.float32),
        pltpu.VMEM((D, B), jnp.float32),
        pltpu.VMEM((L,), jnp.float32),
        pltpu.VMEM((L,), jnp.float32),
    ],
)(_sc_dist_body)


def _dense_body(cpl_ref, ctf_ref, eppt_ref, tpl_ref, tt_ref, w_ref, l1m_ref,
                acc1_ref, acc2_ref, scal_ref, closs_ref, rn_ref):
    x = cpl_ref[...]
    ct = ctf_ref[...]
    bce = (jnp.maximum(x, 0.0) - x * ct
           + jnp.log(1.0 + jnp.exp(-jnp.abs(x))))
    closs_ref[...] = jnp.mean(bce, axis=0, keepdims=True)
    entropy = jnp.mean(bce)

    def rn_body(d, _):
        xd = eppt_ref[d]
        na = jnp.sqrt(jnp.sum(xd * xd, axis=1, keepdims=True))
        rn_ref[d] = 1.0 / jnp.maximum(na, EPS)
        return 0
    lax.fori_loop(0, D, rn_body, 0)

    def cos_body(k, carry):
        ap, ac = carry
        d = k // D
        e = k - d * D
        xd = eppt_ref[d]
        xe = eppt_ref[e]
        rnd = rn_ref[d]
        rne = rn_ref[e]
        s_de = jnp.sum(xd * xe, axis=1, keepdims=True)
        ap = ap + jnp.sum(s_de * rnd * rne)
        s_c = jnp.sum(xd[C:, :] * xe[:C, :], axis=1, keepdims=True)
        ac = ac + jnp.sum(s_c * rnd[C:] * rne[:C])
        return ap, ac
    accp, accc = lax.fori_loop(0, D * D, cos_body, (0.0, 0.0))
    ortho_p = accp / (D * C * 2) - 1.0
    ortho_c = accc / (D * C)

    tl = tpl_ref[...]
    m = jnp.max(tl, axis=1, keepdims=True)
    lse = m + jnp.log(jnp.sum(jnp.exp(tl - m), axis=1, keepdims=True))
    lbl = tt_ref[...]
    onehot = lax.broadcasted_iota(jnp.int32, (B, NC), 1) == lbl
    picked = jnp.sum(jnp.where(onehot, tl, 0.0), axis=1, keepdims=True)
    target_loss = jnp.mean(lse - picked)

    l1 = jnp.sum(jnp.abs(w_ref[...] * l1m_ref[...]))

    clst = MAX_DIST - jnp.sum(acc1_ref[...]) / (B * C)
    sep = MAX_DIST - jnp.sum(acc2_ref[...]) / (B * C)

    summed = (COEFS["crs_ent"] * entropy + COEFS["clst"] * clst
              + COEFS["sep"] * sep + COEFS["l1"] * l1
              + COEFS["ortho_p"] * ortho_p + COEFS["ortho_c"] * ortho_c)
    total = target_loss + ALPHA * summed

    li = lax.broadcasted_iota(jnp.int32, (1, 16), 1)
    vals = [target_loss, summed, total, entropy, clst, sep, l1,
            ortho_p, ortho_c]
    out = jnp.zeros((1, 16), jnp.float32)
    for i, val in enumerate(vals):
        out = jnp.where(li == i, val, out)
    scal_ref[...] = out


@jax.jit
def kernel(concepts_pred_logits, concepts_true, min_distances, proto_presence,
           target_pred_logits, target_true, encoder_proto_presence,
           last_layer_weight, l1_mask):
    ctf = concepts_true.astype(jnp.float32)
    eppt = jnp.transpose(encoder_proto_presence, (2, 0, 1))
    w1t = jnp.concatenate([ctf, 1.0 - ctf], axis=1).T.reshape(-1)
    mdt = min_distances.T
    ppf = proto_presence.reshape(-1)
    tt2 = target_true.astype(jnp.int32).reshape(B, 1)

    acc1, acc2 = _sc_dist(mdt, ppf, w1t)

    scal, closs = pl.pallas_call(
        _dense_body,
        out_shape=[
            jax.ShapeDtypeStruct((1, 16), jnp.float32),
            jax.ShapeDtypeStruct((1, C), jnp.float32),
        ],
        scratch_shapes=[pltpu.VMEM((D, R2, 1), jnp.float32)],
    )(concepts_pred_logits, ctf, eppt, target_pred_logits, tt2,
      last_layer_weight, l1_mask, acc1, acc2)

    return (scal[0, 0], closs[0], scal[0, 1], scal[0, 2], scal[0, 3],
            scal[0, 4], scal[0, 5], scal[0, 6], scal[0, 7], scal[0, 8])


# speedup vs baseline: 248.6214x; 1.9468x over previous
import functools

import jax
import jax.numpy as jnp
from jax import lax
from jax.experimental import pallas as pl
from jax.experimental.pallas import tpu as pltpu
from jax.experimental.pallas import tpu_sc as plsc

B = 128
C = 112
P = 256
D = 10
NC = 200
MAX_DIST = 128.0
ALPHA = 1.0
COEFS = {"crs_ent": 1.0, "clst": 0.8, "sep": -0.08, "l1": 1e-4,
         "ortho_p": 1.0, "ortho_c": 1.0}
EPS = 1e-8
R2 = 2 * C
L = 16
NW = 32
RPW = R2 // NW
NBLK = P // L
NBB = B // L
BIGF = 3.0e38
NEGF = -3.0e38


def _sc_dist_body(mdt_hbm, ppf_hbm, wt_hbm, out1_hbm, out2_hbm,
                  mdv, ppv, wv, sv, a1v, a2v):
    cid = lax.axis_index("c")
    sid = lax.axis_index("s")
    wid = sid * 2 + cid
    pltpu.sync_copy(mdt_hbm, mdv)
    for d in range(D):
        pltpu.sync_copy(
            ppf_hbm.at[pl.ds(d * (R2 * P) + wid * (RPW * P), RPW * P)],
            ppv.at[pl.ds(d * (RPW * P), RPW * P)])
    pltpu.sync_copy(wt_hbm.at[pl.ds(wid * RPW * B, RPW * B)], wv)

    iota = lax.iota(jnp.int32, L)

    def row_body(i, accs):
        acc1, acc2 = accs
        pbase = i * P
        s_chunks = []
        for blk in range(NBLK):
            off = pbase + blk * L
            acc = ppv[pl.ds(off, L)]
            for d in range(1, D):
                acc = acc + ppv[pl.ds(d * (RPW * P) + off, L)]
            s_chunks.append(acc)

        top_idx = []
        for _ in range(D):
            m = s_chunks[0]
            for blk in range(1, NBLK):
                m = jnp.maximum(m, s_chunks[blk])
            gm = jnp.max(m)
            cand = jnp.where(s_chunks[0] == gm, iota, jnp.int32(P))
            for blk in range(1, NBLK):
                cblk = jnp.where(s_chunks[blk] == gm, iota + blk * L,
                                 jnp.int32(P))
                cand = jnp.minimum(cand, cblk)
            idx = jnp.min(cand)
            top_idx.append(idx)
            for blk in range(NBLK):
                hit = (iota + blk * L) == idx
                s_chunks[blk] = jnp.where(hit, jnp.float32(NEGF),
                                          s_chunks[blk])

        def patch_body(k, _):
            b0 = k * L
            for j in range(D):
                sv[j, pl.ds(b0, L)] = mdv[top_idx[j], pl.ds(b0, L)]
                mdv[top_idx[j], pl.ds(b0, L)] = jnp.full((L,), BIGF,
                                                         jnp.float32)
            return 0
        lax.fori_loop(0, NBB, patch_body, 0)

        def bb_body(k, accs2):
            a1, a2 = accs2
            b0 = k * L

            def p_body(q, mn_c):
                p0 = q * 8
                for j in range(8):
                    mn_c = jnp.minimum(mn_c, mdv[p0 + j, pl.ds(b0, L)])
                return mn_c
            mn = lax.fori_loop(0, P // 8, p_body,
                               jnp.full((L,), BIGF, jnp.float32))
            m2 = jnp.maximum(MAX_DIST - mn, 0.0)

            mt = sv[0, pl.ds(b0, L)]
            for j in range(1, D):
                mt = jnp.minimum(mt, sv[j, pl.ds(b0, L)])
            m1 = jnp.maximum(MAX_DIST - mt, 0.0)

            w = wv[pl.ds(i * B + b0, L)]
            return (a1 + w * m1, a2 + (1.0 - w) * m2)
        acc1, acc2 = lax.fori_loop(0, NBB, bb_body, (acc1, acc2))

        def rest_body(k, _):
            b0 = k * L
            for j in range(D):
                mdv[top_idx[j], pl.ds(b0, L)] = sv[j, pl.ds(b0, L)]
            return 0
        lax.fori_loop(0, NBB, rest_body, 0)
        return (acc1, acc2)

    zero = jnp.zeros((L,), jnp.float32)
    acc1, acc2 = lax.fori_loop(0, RPW, row_body, (zero, zero))
    a1v[...] = acc1
    a2v[...] = acc2
    pltpu.sync_copy(a1v, out1_hbm.at[wid])
    pltpu.sync_copy(a2v, out2_hbm.at[wid])


_sc_dist = functools.partial(
    pl.kernel,
    mesh=plsc.VectorSubcoreMesh(core_axis_name="c", subcore_axis_name="s"),
    compiler_params=pltpu.CompilerParams(needs_layout_passes=False),
    out_type=[jax.ShapeDtypeStruct((NW, L), jnp.float32),
              jax.ShapeDtypeStruct((NW, L), jnp.float32)],
    scratch_types=[
        pltpu.VMEM((P, B), jnp.float32),
        pltpu.VMEM((RPW * P * D,), jnp.float32),
        pltpu.VMEM((RPW * B,), jnp.float32),
        pltpu.VMEM((D, B), jnp.float32),
        pltpu.VMEM((L,), jnp.float32),
        pltpu.VMEM((L,), jnp.float32),
    ],
)(_sc_dist_body)


def _dense_body(cpl_ref, ctf_ref, eppt_ref, tpl_ref, tt_ref, w_ref, l1m_ref,
                acc1_ref, acc2_ref, scal_ref, closs_ref, rn_ref):
    x = cpl_ref[...]
    ct = ctf_ref[...]
    bce = (jnp.maximum(x, 0.0) - x * ct
           + jnp.log(1.0 + jnp.exp(-jnp.abs(x))))
    closs_ref[...] = jnp.mean(bce, axis=0, keepdims=True)
    entropy = jnp.mean(bce)

    def rn_body(d, _):
        xd = eppt_ref[d]
        na = jnp.sqrt(jnp.sum(xd * xd, axis=1, keepdims=True))
        rn_ref[d] = 1.0 / jnp.maximum(na, EPS)
        return 0
    lax.fori_loop(0, D, rn_body, 0)

    def u_body(d, u):
        return u + eppt_ref[d] * rn_ref[d]
    u = lax.fori_loop(0, D, u_body, jnp.zeros((R2, P), jnp.float32))
    accp = jnp.sum(u * u)
    accc = jnp.sum(u[C:, :] * u[:C, :])
    ortho_p = accp / (D * C * 2) - 1.0
    ortho_c = accc / (D * C)

    tl = tpl_ref[...]
    m = jnp.max(tl, axis=1, keepdims=True)
    lse = m + jnp.log(jnp.sum(jnp.exp(tl - m), axis=1, keepdims=True))
    lbl = tt_ref[...]
    onehot = lax.broadcasted_iota(jnp.int32, (B, NC), 1) == lbl
    picked = jnp.sum(jnp.where(onehot, tl, 0.0), axis=1, keepdims=True)
    target_loss = jnp.mean(lse - picked)

    l1 = jnp.sum(jnp.abs(w_ref[...] * l1m_ref[...]))

    clst = MAX_DIST - jnp.sum(acc1_ref[...]) / (B * C)
    sep = MAX_DIST - jnp.sum(acc2_ref[...]) / (B * C)

    summed = (COEFS["crs_ent"] * entropy + COEFS["clst"] * clst
              + COEFS["sep"] * sep + COEFS["l1"] * l1
              + COEFS["ortho_p"] * ortho_p + COEFS["ortho_c"] * ortho_c)
    total = target_loss + ALPHA * summed

    li = lax.broadcasted_iota(jnp.int32, (1, 16), 1)
    vals = [target_loss, summed, total, entropy, clst, sep, l1,
            ortho_p, ortho_c]
    out = jnp.zeros((1, 16), jnp.float32)
    for i, val in enumerate(vals):
        out = jnp.where(li == i, val, out)
    scal_ref[...] = out


@jax.jit
def kernel(concepts_pred_logits, concepts_true, min_distances, proto_presence,
           target_pred_logits, target_true, encoder_proto_presence,
           last_layer_weight, l1_mask):
    ctf = concepts_true.astype(jnp.float32)
    eppt = jnp.transpose(encoder_proto_presence, (2, 0, 1))
    w1t = jnp.concatenate([ctf, 1.0 - ctf], axis=1).T.reshape(-1)
    mdt = min_distances.T
    ppf = jnp.transpose(proto_presence, (2, 0, 1)).reshape(-1)
    tt2 = target_true.astype(jnp.int32).reshape(B, 1)

    acc1, acc2 = _sc_dist(mdt, ppf, w1t)

    scal, closs = pl.pallas_call(
        _dense_body,
        out_shape=[
            jax.ShapeDtypeStruct((1, 16), jnp.float32),
            jax.ShapeDtypeStruct((1, C), jnp.float32),
        ],
        scratch_shapes=[pltpu.VMEM((D, R2, 1), jnp.float32)],
    )(concepts_pred_logits, ctf, eppt, target_pred_logits, tt2,
      last_layer_weight, l1_mask, acc1, acc2)

    return (scal[0, 0], closs[0], scal[0, 1], scal[0, 2], scal[0, 3],
            scal[0, 4], scal[0, 5], scal[0, 6], scal[0, 7], scal[0, 8])
